# baseline (device time: 7034 ns/iter reference)
import jax
import jax.numpy as jnp
from jax import lax
from jax.experimental import pallas as pl
from jax.experimental.pallas import tpu as pltpu

N_CHUNKS = 4


def kernel(x, dy, gamma):
    m, d = x.shape
    chunk = m // N_CHUNKS

    def body(x_hbm, dy_hbm, gamma_hbm, out_ref, xv_ref, dyv_ref,
             acc_ref, recv_ref, sems_x, sem_dy, send_sem, recv_sem, out_sem):
        my_x = lax.axis_index("x")
        my_y = lax.axis_index("y")
        my_z = lax.axis_index("z")
        partner = (my_x, my_y, 1 - my_z)

        barrier_sem = pltpu.get_barrier_semaphore()
        pl.semaphore_signal(
            barrier_sem, inc=1,
            device_id=partner, device_id_type=pl.DeviceIdType.MESH,
        )

        cp_dy = pltpu.make_async_copy(dy_hbm, dyv_ref, sem_dy)
        cp_dy.start()
        cp_x = []
        for k in range(N_CHUNKS):
            rows = pl.ds(k * chunk, chunk)
            cp = pltpu.make_async_copy(
                x_hbm.at[rows], xv_ref.at[rows], sems_x.at[k])
            cp.start()
            cp_x.append(cp)

        cp_dy.wait()
        acc_ref[1, :] = jnp.sum(dyv_ref[...], axis=0)

        dg = jnp.zeros((d,), jnp.float32)
        for k in range(N_CHUNKS):
            cp_x[k].wait()
            rows = pl.ds(k * chunk, chunk)
            xv = xv_ref[rows, :]
            dyv = dyv_ref[rows, :]
            mu = jnp.mean(xv, axis=1, keepdims=True)
            xc = xv - mu
            var = jnp.mean(xc * xc, axis=1, keepdims=True)
            rstd = lax.rsqrt(var + 1e-5)
            xhat = xc * rstd
            dg = dg + jnp.sum(dyv * xhat, axis=0)
        acc_ref[0, :] = dg

        pl.semaphore_wait(barrier_sem, 1)
        rdma = pltpu.make_async_remote_copy(
            src_ref=acc_ref,
            dst_ref=recv_ref,
            send_sem=send_sem,
            recv_sem=recv_sem,
            device_id=partner,
            device_id_type=pl.DeviceIdType.MESH,
        )
        rdma.start()
        rdma.wait_recv()

        recv_ref[...] += acc_ref[...]
        cp_out = pltpu.make_async_copy(recv_ref, out_ref, out_sem)
        cp_out.start()
        rdma.wait_send()
        cp_out.wait()

    return pl.pallas_call(
        body,
        out_shape=jax.ShapeDtypeStruct((2, d), jnp.float32),
        in_specs=[pl.BlockSpec(memory_space=pl.ANY)] * 3,
        out_specs=pl.BlockSpec(memory_space=pltpu.MemorySpace.HBM),
        scratch_shapes=[
            pltpu.VMEM((m, d), jnp.float32),
            pltpu.VMEM((m, d), jnp.float32),
            pltpu.VMEM((2, d), jnp.float32),
            pltpu.VMEM((2, d), jnp.float32),
            pltpu.SemaphoreType.DMA((N_CHUNKS,)),
            pltpu.SemaphoreType.DMA,
            pltpu.SemaphoreType.DMA,
            pltpu.SemaphoreType.DMA,
            pltpu.SemaphoreType.DMA,
        ],
        compiler_params=pltpu.CompilerParams(collective_id=0),
    )(
        pltpu.with_memory_space_constraint(x, pltpu.MemorySpace.HBM),
        pltpu.with_memory_space_constraint(dy, pltpu.MemorySpace.HBM),
        pltpu.with_memory_space_constraint(gamma, pltpu.MemorySpace.HBM),
    )
